# baseline (speedup 1.0000x reference)
import functools

import jax
import jax.numpy as jnp
from jax import lax
from jax.experimental import pallas as pl
from jax.experimental.pallas import tpu as pltpu
from jax.experimental.pallas import tpu_sc as plsc

NC = 2
NS = 16
NW = NC * NS

NBUF = 8


def _emb_lookup(x, table):
    batch, hist = x.shape
    d = table.shape[1]
    bw = batch // NW
    mesh = plsc.VectorSubcoreMesh(core_axis_name="c", subcore_axis_name="s")

    @functools.partial(
        pl.kernel,
        out_type=jax.ShapeDtypeStruct((batch, hist, d), jnp.float32),
        mesh=mesh,
        scratch_types=(
            [pltpu.VMEM((bw, hist), jnp.int32)]
            + [pltpu.VMEM((hist, d), jnp.float32)] * NBUF
            + [pltpu.SemaphoreType.DMA] * NBUF
        ),
        compiler_params=pltpu.CompilerParams(use_tc_tiling_on_sc=False),
    )
    def k(x_hbm, table_hbm, out_hbm, idx_v, *bufs_sems):
        bufs = bufs_sems[:NBUF]
        sems = bufs_sems[NBUF:]
        wid = lax.axis_index("s") * NC + lax.axis_index("c")
        base = wid * bw

        pltpu.sync_copy(x_hbm.at[pl.ds(base, bw)], idx_v)

        for b in range(NBUF):
            pltpu.async_copy(table_hbm.at[idx_v.at[b]], bufs[b], sems[b])

        @pl.loop(0, bw, step=NBUF)
        def _(j):
            for b in range(NBUF):
                jj = j + b
                pltpu.make_async_copy(
                    table_hbm.at[idx_v.at[jj]], bufs[b], sems[b]
                ).wait()
                pltpu.sync_copy(bufs[b], out_hbm.at[base + jj])

                @pl.when(jj + NBUF < bw)
                def _():
                    pltpu.async_copy(
                        table_hbm.at[idx_v.at[jj + NBUF]], bufs[b], sems[b]
                    )

    return k(x, table)


def kernel(x, table):
    return _emb_lookup(x.astype(jnp.int32), table)


# speedup vs baseline: 1.0978x; 1.0978x over previous
import functools

import jax
import jax.numpy as jnp
from jax import lax
from jax.experimental import pallas as pl
from jax.experimental.pallas import tpu as pltpu
from jax.experimental.pallas import tpu_sc as plsc

NC = 2
NS = 16
NW = NC * NS

NBUF = 4
LANES = 16


def _emb_lookup(x, table):
    batch, hist = x.shape
    dp = table.shape[1]
    d = dp // 2
    bw = batch // NW
    mesh = plsc.VectorSubcoreMesh(core_axis_name="c", subcore_axis_name="s")

    @functools.partial(
        pl.kernel,
        out_type=jax.ShapeDtypeStruct((batch, hist, d), jnp.float32),
        mesh=mesh,
        scratch_types=(
            [pltpu.VMEM((bw, hist), jnp.int32)]
            + [pltpu.VMEM((hist, dp), jnp.float32)] * NBUF
            + [pltpu.VMEM((hist, d), jnp.float32)]
            + [pltpu.SemaphoreType.DMA] * NBUF
        ),
        compiler_params=pltpu.CompilerParams(use_tc_tiling_on_sc=True),
    )
    def k(x_hbm, table_hbm, out_hbm, idx_v, *bufs_sems):
        bufs = bufs_sems[:NBUF]
        stage = bufs_sems[NBUF]
        sems = bufs_sems[NBUF + 1:]
        wid = lax.axis_index("s") * NC + lax.axis_index("c")
        base = wid * bw

        pltpu.sync_copy(x_hbm.at[pl.ds(base, bw)], idx_v)

        for b in range(NBUF):
            pltpu.async_copy(table_hbm.at[idx_v.at[b]], bufs[b], sems[b])

        @pl.loop(0, bw, step=NBUF)
        def _(j):
            for b in range(NBUF):
                jj = j + b
                pltpu.make_async_copy(
                    table_hbm.at[idx_v.at[jj]], bufs[b], sems[b]
                ).wait()

                @pl.loop(0, hist)
                def _(h):
                    for c in range(d // LANES):
                        stage[h, pl.ds(c * LANES, LANES)] = bufs[b][
                            h, pl.ds(c * LANES, LANES)
                        ]

                pltpu.sync_copy(stage, out_hbm.at[base + jj])

                @pl.when(jj + NBUF < bw)
                def _():
                    pltpu.async_copy(
                        table_hbm.at[idx_v.at[jj + NBUF]], bufs[b], sems[b]
                    )

    return k(x, table)


def kernel(x, table):
    d = table.shape[1]
    table_p = jnp.pad(table, ((0, 0), (0, d)))
    return _emb_lookup(x.astype(jnp.int32), table_p)


# speedup vs baseline: 1.0995x; 1.0016x over previous
import functools

import jax
import jax.numpy as jnp
from jax import lax
from jax.experimental import pallas as pl
from jax.experimental.pallas import tpu as pltpu
from jax.experimental.pallas import tpu_sc as plsc

NC = 2
NS = 16
NW = NC * NS

NBUF = 4
LANES = 16


def _emb_lookup(x, table):
    batch, hist = x.shape
    dp = table.shape[1]
    d = dp // 2
    bw = batch // NW
    mesh = plsc.VectorSubcoreMesh(core_axis_name="c", subcore_axis_name="s")

    @functools.partial(
        pl.kernel,
        out_type=jax.ShapeDtypeStruct((batch, hist, d), jnp.float32),
        mesh=mesh,
        scratch_types=(
            [pltpu.VMEM((bw, hist), jnp.int32)]
            + [pltpu.VMEM((hist, dp), jnp.float32)] * NBUF
            + [pltpu.VMEM((hist, d), jnp.float32)] * 2
            + [pltpu.SemaphoreType.DMA] * (NBUF + 2)
        ),
        compiler_params=pltpu.CompilerParams(use_tc_tiling_on_sc=True),
    )
    def k(x_hbm, table_hbm, out_hbm, idx_v, *bufs_sems):
        bufs = bufs_sems[:NBUF]
        stages = bufs_sems[NBUF:NBUF + 2]
        sems = bufs_sems[NBUF + 2:NBUF + 2 + NBUF]
        wsems = bufs_sems[NBUF + 2 + NBUF:]
        wid = lax.axis_index("s") * NC + lax.axis_index("c")
        base = wid * bw

        pltpu.sync_copy(x_hbm.at[pl.ds(base, bw)], idx_v)

        for b in range(NBUF):
            pltpu.async_copy(table_hbm.at[idx_v.at[b]], bufs[b], sems[b])

        @pl.loop(0, bw, step=NBUF)
        def _(j):
            for b in range(NBUF):
                jj = j + b
                s = b % 2
                pltpu.make_async_copy(
                    table_hbm.at[idx_v.at[jj]], bufs[b], sems[b]
                ).wait()

                @pl.when(jj >= 2)
                def _():
                    pltpu.make_async_copy(
                        stages[s], out_hbm.at[base], wsems[s]
                    ).wait()

                @pl.loop(0, hist)
                def _(h):
                    for c in range(d // LANES):
                        stages[s][h, pl.ds(c * LANES, LANES)] = bufs[b][
                            h, pl.ds(c * LANES, LANES)
                        ]

                @pl.when(jj + NBUF < bw)
                def _():
                    pltpu.async_copy(
                        table_hbm.at[idx_v.at[jj + NBUF]], bufs[b], sems[b]
                    )

                pltpu.async_copy(stages[s], out_hbm.at[base + jj], wsems[s])

        for s in range(2):
            pltpu.make_async_copy(stages[s], out_hbm.at[base], wsems[s]).wait()

    return k(x, table)


def kernel(x, table):
    d = table.shape[1]
    table_p = jnp.pad(table, ((0, 0), (0, d)))
    return _emb_lookup(x.astype(jnp.int32), table_p)
